# baseline (speedup 1.0000x reference)
import functools

import jax
import jax.numpy as jnp
from jax import lax
from jax.experimental import pallas as pl
from jax.experimental.pallas import tpu as pltpu
from jax.experimental.pallas import tpu_sc as plsc

_N = 50000
_D_IN = 1280
_D = 256
_NMOL = 512
_INV_SQRT_2 = 0.7071067811865476

_BLOCK = 2048
_NW = 32
_KCH = 13
_CHUNK = _KCH * 128
_NPAD = _NW * _CHUNK
_NACC = _NMOL + 16
_EW = 16


def _s(u):
    return u + u * jnp.tanh(u)


def _dot1(a, w_bf):
    return jax.lax.dot(a.astype(jnp.bfloat16), w_bf,
                       preferred_element_type=jnp.float32)


_NSUB = 2


def _mlp_chain(x, wd, wa1, wb1, wa2, wb2, woa, wob):
    h1 = _s(_dot1(x, wd))
    y1 = _s(_dot1(_s(_dot1(h1, wa1)), wb1))
    h2u = h1 + y1
    y2 = _s(_dot1(_s(_dot1(h2u, wa2)), wb2))
    e16 = _dot1(h2u, woa) + _dot1(y2, wob)
    return jnp.sum(e16, axis=1)


def _mlp_body(x_ref, wd, wa1, wb1, wa2, wb2, woa, wob, e_ref):
    sub = _BLOCK // _NSUB
    for i in range(_NSUB):
        rows = pl.ds(i * sub, sub)
        e_ref[rows] = _mlp_chain(x_ref[rows, :], wd[...], wa1[...],
                                 wb1[...], wa2[...], wb2[...],
                                 woa[...], wob[...])


def _atom_energies(xs, w_bfs):
    grid = pl.cdiv(_N, _BLOCK)
    full = lambda i: (0, 0)
    w_specs = [pl.BlockSpec(w.shape, full) for w in w_bfs]
    return pl.pallas_call(
        _mlp_body,
        grid=(grid,),
        in_specs=[pl.BlockSpec((_BLOCK, _D_IN), lambda i: (i, 0))] + w_specs,
        out_specs=pl.BlockSpec((_BLOCK,), lambda i: (i,)),
        out_shape=jax.ShapeDtypeStruct((_NPAD,), jnp.float32),
        compiler_params=pltpu.CompilerParams(
            dimension_semantics=("arbitrary",),
        ),
    )(xs, *w_bfs)


def _segment_sum_sc(e_pad, b_pad):
    mesh = plsc.VectorSubcoreMesh(core_axis_name="c", subcore_axis_name="s")

    @functools.partial(
        pl.kernel,
        mesh=mesh,
        out_type=jax.ShapeDtypeStruct((2, _NMOL), jnp.float32),
        scratch_types=[
            pltpu.VMEM((_CHUNK,), jnp.float32),
            pltpu.VMEM((_KCH, 128), jnp.int32),
            pltpu.VMEM((_NACC,), jnp.float32),
            pltpu.VMEM_SHARED((_NACC,), jnp.float32),
            pltpu.SemaphoreType.DMA,
        ],
    )
    def seg(e_hbm, b_hbm, out_hbm, e_v, idx_v, stage_v, acc_sh, sem):
        c = lax.axis_index("c")
        s = lax.axis_index("s")
        w = c * 16 + s
        cp_e = pltpu.async_copy(e_hbm.at[pl.ds(w * _CHUNK, _CHUNK)], e_v, sem)
        cp_b = pltpu.async_copy(b_hbm.at[w], idx_v, sem)
        for i in range(_NACC // 16):
            stage_v[pl.ds(i * 16, 16)] = jnp.zeros((16,), jnp.float32)

        @pl.when(s == 0)
        def _():
            pltpu.sync_copy(stage_v, acc_sh)

        cp_e.wait()
        cp_b.wait()
        plsc.subcore_barrier()
        cps = [
            pltpu.async_copy(e_v.at[pl.ds(j * 128, 128)],
                             acc_sh.at[idx_v.at[j]], sem, add=True)
            for j in range(_KCH)
        ]
        for cp in cps:
            cp.wait()
        plsc.subcore_barrier()

        @pl.when(s == 0)
        def _():
            pltpu.sync_copy(acc_sh.at[pl.ds(0, _NMOL)], out_hbm.at[c])

    return seg(e_pad, b_pad)


def kernel(xs_E_cat, batch, W_dense, W_r1a, W_r1b, W_r2a, W_r2b, W_out):
    c = _INV_SQRT_2
    wo_col = W_out.reshape(_D, 1)
    pad_wo = jnp.zeros((_D, _EW - 1), jnp.float32)
    w_bfs = [
        w.astype(jnp.bfloat16)
        for w in (
            W_dense * 0.5,
            W_r1a * 0.5,
            W_r1b * 0.5,
            W_r2a * (c * 0.5),
            W_r2b * 0.5,
            jnp.concatenate([wo_col * (c * c), pad_wo], axis=1),
            jnp.concatenate([wo_col * c, pad_wo], axis=1),
        )
    ]
    e_pad = _atom_energies(xs_E_cat, w_bfs)
    b_pad = jnp.pad(batch.astype(jnp.int32), (0, _NPAD - _N),
                    constant_values=_NMOL).reshape(_NW, _KCH, 128)
    parts = _segment_sum_sc(e_pad, b_pad)
    return parts[0] + parts[1]


# speedup vs baseline: 1.0032x; 1.0032x over previous
import functools

import jax
import jax.numpy as jnp
from jax import lax
from jax.experimental import pallas as pl
from jax.experimental.pallas import tpu as pltpu
from jax.experimental.pallas import tpu_sc as plsc

_N = 50000
_D_IN = 1280
_D = 256
_NMOL = 512
_INV_SQRT_2 = 0.7071067811865476

_BLOCK = 1024
_NW = 32
_KCH = 13
_CHUNK = _KCH * 128
_NPAD = _NW * _CHUNK
_NACC = _NMOL + 16
_EW = 16


def _s(u):
    return u + u * jnp.tanh(u)


def _dot1(a, w_bf):
    return jax.lax.dot(a.astype(jnp.bfloat16), w_bf,
                       preferred_element_type=jnp.float32)


_NSUB = 2


def _mlp_chain(x, wd, wa1, wb1, wa2, wb2, woa, wob):
    f = jnp.float32
    h1 = _s(_dot1(x, wd))
    y1 = _s(_dot1(_s(_dot1(h1, wa1)), wb1))
    h2b = (h1 + y1).astype(jnp.bfloat16)
    y2 = _s(_dot1(_s(jax.lax.dot(h2b, wa2, preferred_element_type=f)), wb2))
    e16 = (jax.lax.dot(h2b, woa, preferred_element_type=f)
           + _dot1(y2, wob))
    return jnp.sum(e16, axis=1)


def _mlp_body(x_ref, wd, wa1, wb1, wa2, wb2, woa, wob, e_ref):
    sub = _BLOCK // _NSUB
    for i in range(_NSUB):
        rows = pl.ds(i * sub, sub)
        e_ref[rows] = _mlp_chain(x_ref[rows, :], wd[...], wa1[...],
                                 wb1[...], wa2[...], wb2[...],
                                 woa[...], wob[...])


def _atom_energies(xs, w_bfs):
    grid = pl.cdiv(_N, _BLOCK)
    full = lambda i: (0, 0)
    w_specs = [pl.BlockSpec(w.shape, full) for w in w_bfs]
    return pl.pallas_call(
        _mlp_body,
        grid=(grid,),
        in_specs=[pl.BlockSpec((_BLOCK, _D_IN), lambda i: (i, 0))] + w_specs,
        out_specs=pl.BlockSpec((_BLOCK,), lambda i: (i,)),
        out_shape=jax.ShapeDtypeStruct((_NPAD,), jnp.float32),
        compiler_params=pltpu.CompilerParams(
            dimension_semantics=("arbitrary",),
        ),
    )(xs, *w_bfs)


def _segment_sum_sc(e_pad, b_pad):
    mesh = plsc.VectorSubcoreMesh(core_axis_name="c", subcore_axis_name="s")

    @functools.partial(
        pl.kernel,
        mesh=mesh,
        out_type=jax.ShapeDtypeStruct((2, _NMOL), jnp.float32),
        scratch_types=[
            pltpu.VMEM((_CHUNK,), jnp.float32),
            pltpu.VMEM((_KCH, 128), jnp.int32),
            pltpu.VMEM((_NACC,), jnp.float32),
            pltpu.VMEM_SHARED((_NACC,), jnp.float32),
            pltpu.SemaphoreType.DMA,
        ],
    )
    def seg(e_hbm, b_hbm, out_hbm, e_v, idx_v, stage_v, acc_sh, sem):
        c = lax.axis_index("c")
        s = lax.axis_index("s")
        w = c * 16 + s
        cp_e = pltpu.async_copy(e_hbm.at[pl.ds(w * _CHUNK, _CHUNK)], e_v, sem)
        cp_b = pltpu.async_copy(b_hbm.at[w], idx_v, sem)
        for i in range(_NACC // 16):
            stage_v[pl.ds(i * 16, 16)] = jnp.zeros((16,), jnp.float32)

        @pl.when(s == 0)
        def _():
            pltpu.sync_copy(stage_v, acc_sh)

        cp_e.wait()
        cp_b.wait()
        plsc.subcore_barrier()
        cps = [
            pltpu.async_copy(e_v.at[pl.ds(j * 128, 128)],
                             acc_sh.at[idx_v.at[j]], sem, add=True)
            for j in range(_KCH)
        ]
        for cp in cps:
            cp.wait()
        plsc.subcore_barrier()

        @pl.when(s == 0)
        def _():
            pltpu.sync_copy(acc_sh.at[pl.ds(0, _NMOL)], out_hbm.at[c])

    return seg(e_pad, b_pad)


def kernel(xs_E_cat, batch, W_dense, W_r1a, W_r1b, W_r2a, W_r2b, W_out):
    c = _INV_SQRT_2
    wo_col = W_out.reshape(_D, 1)
    pad_wo = jnp.zeros((_D, _EW - 1), jnp.float32)
    w_bfs = [
        w.astype(jnp.bfloat16)
        for w in (
            W_dense * 0.5,
            W_r1a * 0.5,
            W_r1b * 0.5,
            W_r2a * (c * 0.5),
            W_r2b * 0.5,
            jnp.concatenate([wo_col * (c * c), pad_wo], axis=1),
            jnp.concatenate([wo_col * c, pad_wo], axis=1),
        )
    ]
    e_pad = _atom_energies(xs_E_cat, w_bfs)
    b_pad = jnp.pad(batch.astype(jnp.int32), (0, _NPAD - _N),
                    constant_values=_NMOL).reshape(_NW, _KCH, 128)
    parts = _segment_sum_sc(e_pad, b_pad)
    return parts[0] + parts[1]


# speedup vs baseline: 1.0086x; 1.0053x over previous
import functools

import jax
import jax.numpy as jnp
from jax import lax
from jax.experimental import pallas as pl
from jax.experimental.pallas import tpu as pltpu
from jax.experimental.pallas import tpu_sc as plsc

_N = 50000
_D_IN = 1280
_D = 256
_NMOL = 512
_INV_SQRT_2 = 0.7071067811865476

_BLOCK = 1024
_NW = 32
_KCH = 13
_CHUNK = _KCH * 128
_NPAD = _NW * _CHUNK
_NACC = _NMOL + 16
_EW = 16


def _s(u):
    return u + u * jnp.tanh(u)


def _dot1(a, w_bf):
    return jax.lax.dot(a.astype(jnp.bfloat16), w_bf,
                       preferred_element_type=jnp.float32)


_NSUB = 2


def _mlp_chain(x, wd, wa1, wb1, wa2, wb2, woa, wob):
    f = jnp.float32
    h1 = _s(_dot1(x, wd))
    y1 = _s(_dot1(_s(_dot1(h1, wa1)), wb1))
    h2b = (h1 + y1).astype(jnp.bfloat16)
    y2 = _s(_dot1(_s(jax.lax.dot(h2b, wa2, preferred_element_type=f)), wb2))
    e16 = (jax.lax.dot(h2b, woa, preferred_element_type=f)
           + _dot1(y2, wob))
    return jnp.sum(e16, axis=1)


def _mlp_body(x_ref, wd, wa1, wb1, wa2, wb2, woa, wob, e_ref):
    sub = _BLOCK // _NSUB
    for i in range(_NSUB):
        rows = pl.ds(i * sub, sub)
        e_ref[rows] = _mlp_chain(x_ref[rows, :], wd[...], wa1[...],
                                 wb1[...], wa2[...], wb2[...],
                                 woa[...], wob[...])


def _atom_energies(xs, w_bfs):
    grid = pl.cdiv(_N, _BLOCK)
    full = lambda i: (0, 0)
    w_specs = [pl.BlockSpec(w.shape, full) for w in w_bfs]
    return pl.pallas_call(
        _mlp_body,
        grid=(grid,),
        in_specs=[pl.BlockSpec((_BLOCK, _D_IN), lambda i: (i, 0))] + w_specs,
        out_specs=pl.BlockSpec((_BLOCK,), lambda i: (i,)),
        out_shape=jax.ShapeDtypeStruct((_NPAD,), jnp.float32),
        compiler_params=pltpu.CompilerParams(
            dimension_semantics=("parallel",),
        ),
    )(xs, *w_bfs)


def _segment_sum_sc(e_pad, b_pad):
    mesh = plsc.VectorSubcoreMesh(core_axis_name="c", subcore_axis_name="s")

    @functools.partial(
        pl.kernel,
        mesh=mesh,
        out_type=jax.ShapeDtypeStruct((2, _NMOL), jnp.float32),
        scratch_types=[
            pltpu.VMEM((_CHUNK,), jnp.float32),
            pltpu.VMEM((_KCH, 128), jnp.int32),
            pltpu.VMEM((_NACC,), jnp.float32),
            pltpu.VMEM_SHARED((_NACC,), jnp.float32),
            pltpu.SemaphoreType.DMA,
        ],
    )
    def seg(e_hbm, b_hbm, out_hbm, e_v, idx_v, stage_v, acc_sh, sem):
        c = lax.axis_index("c")
        s = lax.axis_index("s")
        w = c * 16 + s
        cp_e = pltpu.async_copy(e_hbm.at[pl.ds(w * _CHUNK, _CHUNK)], e_v, sem)
        cp_b = pltpu.async_copy(b_hbm.at[w], idx_v, sem)
        for i in range(_NACC // 16):
            stage_v[pl.ds(i * 16, 16)] = jnp.zeros((16,), jnp.float32)

        @pl.when(s == 0)
        def _():
            pltpu.sync_copy(stage_v, acc_sh)

        cp_e.wait()
        cp_b.wait()
        plsc.subcore_barrier()
        cps = [
            pltpu.async_copy(e_v.at[pl.ds(j * 128, 128)],
                             acc_sh.at[idx_v.at[j]], sem, add=True)
            for j in range(_KCH)
        ]
        for cp in cps:
            cp.wait()
        plsc.subcore_barrier()

        @pl.when(s == 0)
        def _():
            pltpu.sync_copy(acc_sh.at[pl.ds(0, _NMOL)], out_hbm.at[c])

    return seg(e_pad, b_pad)


def kernel(xs_E_cat, batch, W_dense, W_r1a, W_r1b, W_r2a, W_r2b, W_out):
    c = _INV_SQRT_2
    wo_col = W_out.reshape(_D, 1)
    pad_wo = jnp.zeros((_D, _EW - 1), jnp.float32)
    w_bfs = [
        w.astype(jnp.bfloat16)
        for w in (
            W_dense * 0.5,
            W_r1a * 0.5,
            W_r1b * 0.5,
            W_r2a * (c * 0.5),
            W_r2b * 0.5,
            jnp.concatenate([wo_col * (c * c), pad_wo], axis=1),
            jnp.concatenate([wo_col * c, pad_wo], axis=1),
        )
    ]
    e_pad = _atom_energies(xs_E_cat, w_bfs)
    b_pad = jnp.pad(batch.astype(jnp.int32), (0, _NPAD - _N),
                    constant_values=_NMOL).reshape(_NW, _KCH, 128)
    parts = _segment_sum_sc(e_pad, b_pad)
    return parts[0] + parts[1]
